# baseline (speedup 1.0000x reference)
import functools

import jax
import jax.numpy as jnp
from jax import lax
from jax.experimental import pallas as pl
from jax.experimental.pallas import tpu as pltpu
from jax.experimental.pallas import tpu_sc as plsc

N = 10000
E = 160000
D_IN = 256
D_EDGE = 16
H = 512
L = 4
EPS = 1e-5

NP = 10240
NSUB = 16
WIN = 128
NWIN = 80
NBUF = 2
SUP = 2
EPW = NWIN * WIN
EPAD = NSUB * EPW
CHUNK = 128
NCHUNK = H // CHUNK
STRIPE = NP // NSUB

BM = 1024

_mesh = plsc.VectorSubcoreMesh(
    core_axis_name="c", subcore_axis_name="s", num_cores=2, num_subcores=NSUB
)


EW = 32


@functools.partial(
    pl.kernel,
    out_type=jax.ShapeDtypeStruct((NP, 128), jnp.float32),
    mesh=_mesh,
    scratch_types=[
        pltpu.VMEM((NWIN, WIN), jnp.int32),
        pltpu.VMEM((WIN, 128), jnp.float32),
        pltpu.VMEM_SHARED((NP, 128), jnp.float32),
    ],
)
def _sc_edge_stats(ea_hbm, cidx_hbm, zeros_hbm, out_hbm, cidx_v, gbuf_v, acc):
    cid = lax.axis_index("c")
    sid = lax.axis_index("s")

    @pl.when(cid == 0)
    def _():
        pltpu.sync_copy(cidx_hbm.at[sid], cidx_v)
        pltpu.sync_copy(
            zeros_hbm.at[pl.ds(sid * STRIPE, STRIPE)],
            acc.at[pl.ds(sid * STRIPE, STRIPE)],
        )
        plsc.subcore_barrier()

        @pl.loop(0, NWIN)
        def _(j):
            pltpu.sync_copy(ea_hbm.at[pl.ds(sid * EPW + j * WIN, WIN)], gbuf_v)
            pltpu.sync_copy(gbuf_v, acc.at[cidx_v.at[j]], add=True)

        plsc.subcore_barrier()
        pltpu.sync_copy(
            acc.at[pl.ds(sid * STRIPE, STRIPE)],
            out_hbm.at[pl.ds(sid * STRIPE, STRIPE)],
        )


NHALF = NWIN // 2


@functools.partial(
    pl.kernel,
    out_type=jax.ShapeDtypeStruct((NCHUNK, NP, CHUNK), jnp.float32),
    mesh=_mesh,
    scratch_types=[
        pltpu.VMEM((NWIN, WIN), jnp.int32),
        pltpu.VMEM((NWIN, WIN), jnp.int32),
        pltpu.VMEM((WIN, CHUNK), jnp.float32),
        pltpu.VMEM_SHARED((NP, CHUNK), jnp.float32),
    ],
)
def _sc_segsum(m_hbm, ridx_hbm, cidx_hbm, zeros_hbm, out_hbm, ridx_v, cidx_v,
               gbuf_v, acc):
    cid = lax.axis_index("c")
    sid = lax.axis_index("s")

    pltpu.sync_copy(ridx_hbm.at[sid], ridx_v)

    pltpu.sync_copy(cidx_hbm.at[sid], cidx_v)

    for k in range(NCHUNK // 2):
        chunk = 2 * cid + k
        table = m_hbm.at[chunk]

        pltpu.sync_copy(
            zeros_hbm.at[pl.ds(sid * STRIPE, STRIPE)],
            acc.at[pl.ds(sid * STRIPE, STRIPE)],
        )
        plsc.subcore_barrier()

        @pl.loop(0, NWIN)
        def _(j):
            pltpu.sync_copy(table.at[ridx_v.at[j]], gbuf_v)
            pltpu.sync_copy(gbuf_v, acc.at[cidx_v.at[j]], add=True)

        plsc.subcore_barrier()
        pltpu.sync_copy(
            acc.at[pl.ds(sid * STRIPE, STRIPE)],
            out_hbm.at[chunk].at[pl.ds(sid * STRIPE, STRIPE)],
        )
        plsc.subcore_barrier()


def _ln(u, g, b):
    mu = jnp.mean(u, axis=1, keepdims=True)
    var = jnp.mean((u - mu) ** 2, axis=1, keepdims=True)
    return (u - mu) * lax.rsqrt(var + EPS) * g + b


def _enc_body(x_ref, w_ref, b_ref, g_ref, beta_ref, o_ref):
    h = jnp.dot(x_ref[...], w_ref[...], preferred_element_type=jnp.float32)
    h = jnp.maximum(h + b_ref[...], 0.0)
    o_ref[...] = _ln(h, g_ref[...], beta_ref[...])


def _fused_body(h_ref, a_ref, wla_ref, wlb_ref, wra_ref, wrb_ref, bl_ref,
                m_ref, r_ref):
    inv = 1.0 / jnp.maximum(a_ref[:, D_EDGE:D_EDGE + 1], 1.0)
    agg = a_ref[:, :D_EDGE] * inv
    h = h_ref[...]
    m = (jnp.dot(h, wla_ref[...], preferred_element_type=jnp.float32)
         + jnp.dot(agg, wlb_ref[...], preferred_element_type=jnp.float32))
    r = (jnp.dot(h, wra_ref[...], preferred_element_type=jnp.float32)
         + jnp.dot(agg, wrb_ref[...], preferred_element_type=jnp.float32)
         + bl_ref[...])
    for c in range(NCHUNK):
        m_ref[c] = m[:, c * CHUNK:(c + 1) * CHUNK]
    r_ref[...] = r


def _comb_body(h_ref, r_ref, s_ref, a_ref, g_ref, b_ref, o_ref):
    s = jnp.concatenate([s_ref[c] for c in range(NCHUNK)], axis=1)
    inv = 1.0 / jnp.maximum(a_ref[:, D_EDGE:D_EDGE + 1], 1.0)
    u = h_ref[...] + s * inv + r_ref[...]
    o_ref[...] = jnp.maximum(_ln(u, g_ref[...], b_ref[...]), 0.0)


def _read_body(h_ref, w1_ref, b1_ref, w2_ref, b2_ref, o_ref):
    t = jnp.dot(h_ref[...], w1_ref[...], preferred_element_type=jnp.float32)
    t = jnp.maximum(t + b1_ref[...], 0.0)
    o_ref[...] = jnp.dot(t, w2_ref[...], preferred_element_type=jnp.float32) + b2_ref[...]


def _row_block(d):
    return pl.BlockSpec((BM, d), lambda i: (i, 0))


def _full(shape):
    return pl.BlockSpec(shape, lambda i: tuple(0 for _ in shape))


_GRID = NP // BM

_enc_call = pl.pallas_call(
    _enc_body,
    grid=(_GRID,),
    in_specs=[_row_block(D_IN), _full((D_IN, H)), _full((1, H)), _full((1, H)),
              _full((1, H))],
    out_specs=_row_block(H),
    out_shape=jax.ShapeDtypeStruct((NP, H), jnp.float32),
)

_fused_call = pl.pallas_call(
    _fused_body,
    grid=(_GRID,),
    in_specs=[_row_block(H), _row_block(EW), _full((H, H)), _full((D_EDGE, H)),
              _full((H, H)), _full((D_EDGE, H)), _full((1, H))],
    out_specs=[pl.BlockSpec((NCHUNK, BM, CHUNK), lambda i: (0, i, 0)),
               _row_block(H)],
    out_shape=[jax.ShapeDtypeStruct((NCHUNK, NP, CHUNK), jnp.float32),
               jax.ShapeDtypeStruct((NP, H), jnp.float32)],
)

_comb_call = pl.pallas_call(
    _comb_body,
    grid=(_GRID,),
    in_specs=[_row_block(H), _row_block(H),
              pl.BlockSpec((NCHUNK, BM, CHUNK), lambda i: (0, i, 0)),
              _row_block(EW), _full((1, H)), _full((1, H))],
    out_specs=_row_block(H),
    out_shape=jax.ShapeDtypeStruct((NP, H), jnp.float32),
)

_read_call = pl.pallas_call(
    _read_body,
    grid=(_GRID,),
    in_specs=[_row_block(H), _full((H, H // 2)), _full((1, H // 2)),
              _full((H // 2, 128)), _full((1, 128))],
    out_specs=_row_block(128),
    out_shape=jax.ShapeDtypeStruct((NP, 128), jnp.float32),
)


def kernel(x, edge_index, edge_attr, enc_W, enc_b, enc_g, enc_beta,
           Wl, bl, Wr, ln_g, ln_b, rW1, rb1, rW2, rb2):
    f32 = jnp.float32
    pad = EPAD - E
    spread = N + jnp.arange(pad, dtype=jnp.int32) % (NP - N)
    row = jnp.concatenate([edge_index[0], jnp.zeros((pad,), jnp.int32)])
    col = jnp.concatenate([edge_index[1], spread])
    ridx = row.reshape(NSUB, NWIN, WIN)
    cidx = col.reshape(NSUB, NWIN, WIN)
    ea = jnp.concatenate(
        [edge_attr, jnp.ones((E, 1), f32), jnp.zeros((E, 111), f32)], axis=1)
    ea = jnp.concatenate([ea, jnp.zeros((pad, 128), f32)], axis=0)
    xp = jnp.pad(x, ((0, NP - N), (0, 0)))
    z128 = jnp.zeros((NP, CHUNK), f32)

    acc32 = _sc_edge_stats(ea, cidx, z128)[:, :EW]
    h = _enc_call(xp, enc_W, enc_b.reshape(1, H), enc_g.reshape(1, H),
                  enc_beta.reshape(1, H))
    for i in range(L):
        m4, r = _fused_call(h, acc32, Wl[i, :H], Wl[i, H:], Wr[i, :H],
                            Wr[i, H:], bl[i].reshape(1, H))
        s4 = _sc_segsum(m4, ridx, cidx, z128)
        h = _comb_call(h, r, s4, acc32, ln_g[i].reshape(1, H),
                       ln_b[i].reshape(1, H))
    w2p = jnp.pad(rW2, ((0, 0), (0, 126)))
    b2p = jnp.pad(rb2, (0, 126)).reshape(1, 128)
    yp = _read_call(h, rW1, rb1.reshape(1, H // 2), w2p, b2p)
    return yp[:N, :2]


# speedup vs baseline: 1.3194x; 1.3194x over previous
import functools

import jax
import jax.numpy as jnp
from jax import lax
from jax.experimental import pallas as pl
from jax.experimental.pallas import tpu as pltpu
from jax.experimental.pallas import tpu_sc as plsc

N = 10000
E = 160000
D_IN = 256
D_EDGE = 16
H = 512
L = 4
EPS = 1e-5

NP = 10240
NSUB = 16
WIN = 128
NWIN = 79
NBUF = 2
SUP = 2
EPW = NWIN * WIN
EPAD = NSUB * EPW
CHUNK = 128
NCHUNK = H // CHUNK
STRIPE = NP // NSUB

BM = 1024

_mesh = plsc.VectorSubcoreMesh(
    core_axis_name="c", subcore_axis_name="s", num_cores=2, num_subcores=NSUB
)


EW = 32


@functools.partial(
    pl.kernel,
    out_type=jax.ShapeDtypeStruct((NP, 128), jnp.float32),
    mesh=_mesh,
    scratch_types=[
        pltpu.VMEM((NWIN, WIN), jnp.int32),
        pltpu.VMEM((WIN, 128), jnp.float32),
        pltpu.VMEM_SHARED((NP, 128), jnp.float32),
    ],
)
def _sc_edge_stats(ea_hbm, cidx_hbm, zeros_hbm, out_hbm, cidx_v, gbuf_v, acc):
    cid = lax.axis_index("c")
    sid = lax.axis_index("s")

    @pl.when(cid == 0)
    def _():
        pltpu.sync_copy(cidx_hbm.at[sid], cidx_v)
        pltpu.sync_copy(
            zeros_hbm.at[pl.ds(sid * STRIPE, STRIPE)],
            acc.at[pl.ds(sid * STRIPE, STRIPE)],
        )
        plsc.subcore_barrier()

        @pl.loop(0, NWIN)
        def _(j):
            pltpu.sync_copy(ea_hbm.at[pl.ds(sid * EPW + j * WIN, WIN)], gbuf_v)
            pltpu.sync_copy(gbuf_v, acc.at[cidx_v.at[j]], add=True)

        plsc.subcore_barrier()
        pltpu.sync_copy(
            acc.at[pl.ds(sid * STRIPE, STRIPE)],
            out_hbm.at[pl.ds(sid * STRIPE, STRIPE)],
        )


NHALF = NWIN // 2


@functools.partial(
    pl.kernel,
    out_type=jax.ShapeDtypeStruct((NCHUNK, NP, CHUNK), jnp.float32),
    mesh=_mesh,
    scratch_types=[
        pltpu.VMEM((NWIN, WIN), jnp.int32),
        pltpu.VMEM((NWIN, WIN), jnp.int32),
        pltpu.VMEM((WIN, CHUNK), jnp.float32),
        pltpu.VMEM_SHARED((NP, CHUNK), jnp.float32),
    ],
)
def _sc_segsum(m_hbm, ridx_hbm, cidx_hbm, zeros_hbm, out_hbm, ridx_v, cidx_v,
               gbuf_v, acc):
    cid = lax.axis_index("c")
    sid = lax.axis_index("s")

    pltpu.sync_copy(ridx_hbm.at[sid], ridx_v)

    pltpu.sync_copy(cidx_hbm.at[sid], cidx_v)

    for k in range(NCHUNK // 2):
        chunk = 2 * cid + k
        table = m_hbm.at[chunk]

        pltpu.sync_copy(
            zeros_hbm.at[pl.ds(sid * STRIPE, STRIPE)],
            acc.at[pl.ds(sid * STRIPE, STRIPE)],
        )
        plsc.subcore_barrier()

        @pl.loop(0, NWIN)
        def _(j):
            pltpu.sync_copy(table.at[ridx_v.at[j]], gbuf_v)
            pltpu.sync_copy(gbuf_v, acc.at[cidx_v.at[j]], add=True)

        plsc.subcore_barrier()
        pltpu.sync_copy(
            acc.at[pl.ds(sid * STRIPE, STRIPE)],
            out_hbm.at[chunk].at[pl.ds(sid * STRIPE, STRIPE)],
        )
        plsc.subcore_barrier()


def _ln(u, g, b):
    mu = jnp.mean(u, axis=1, keepdims=True)
    var = jnp.mean((u - mu) ** 2, axis=1, keepdims=True)
    return (u - mu) * lax.rsqrt(var + EPS) * g + b


def _enc_body(x_ref, w_ref, b_ref, g_ref, beta_ref, o_ref):
    h = jnp.dot(x_ref[...], w_ref[...], preferred_element_type=jnp.float32)
    h = jnp.maximum(h + b_ref[...], 0.0)
    o_ref[...] = _ln(h, g_ref[...], beta_ref[...])


def _fused_body(h_ref, a_ref, wla_ref, wlb_ref, wra_ref, wrb_ref, bl_ref,
                m_ref, r_ref):
    inv = 1.0 / jnp.maximum(a_ref[:, D_EDGE:D_EDGE + 1], 1.0)
    agg = a_ref[:, :D_EDGE] * inv
    h = h_ref[...]
    m = (jnp.dot(h, wla_ref[...], preferred_element_type=jnp.float32)
         + jnp.dot(agg, wlb_ref[...], preferred_element_type=jnp.float32))
    r = (jnp.dot(h, wra_ref[...], preferred_element_type=jnp.float32)
         + jnp.dot(agg, wrb_ref[...], preferred_element_type=jnp.float32)
         + bl_ref[...])
    for c in range(NCHUNK):
        m_ref[c] = m[:, c * CHUNK:(c + 1) * CHUNK]
    r_ref[...] = r


def _comb_body(h_ref, r_ref, s_ref, a_ref, g_ref, b_ref, o_ref):
    s = jnp.concatenate([s_ref[c] for c in range(NCHUNK)], axis=1)
    inv = 1.0 / jnp.maximum(a_ref[:, D_EDGE:D_EDGE + 1], 1.0)
    u = h_ref[...] + s * inv + r_ref[...]
    o_ref[...] = jnp.maximum(_ln(u, g_ref[...], b_ref[...]), 0.0)


def _read_body(h_ref, w1_ref, b1_ref, w2_ref, b2_ref, o_ref):
    t = jnp.dot(h_ref[...], w1_ref[...], preferred_element_type=jnp.float32)
    t = jnp.maximum(t + b1_ref[...], 0.0)
    o_ref[...] = jnp.dot(t, w2_ref[...], preferred_element_type=jnp.float32) + b2_ref[...]


def _row_block(d):
    return pl.BlockSpec((BM, d), lambda i: (i, 0))


def _full(shape):
    return pl.BlockSpec(shape, lambda i: tuple(0 for _ in shape))


_GRID = NP // BM

_enc_call = pl.pallas_call(
    _enc_body,
    grid=(_GRID,),
    in_specs=[_row_block(D_IN), _full((D_IN, H)), _full((1, H)), _full((1, H)),
              _full((1, H))],
    out_specs=_row_block(H),
    out_shape=jax.ShapeDtypeStruct((NP, H), jnp.float32),
)

_fused_call = pl.pallas_call(
    _fused_body,
    grid=(_GRID,),
    in_specs=[_row_block(H), _row_block(EW), _full((H, H)), _full((D_EDGE, H)),
              _full((H, H)), _full((D_EDGE, H)), _full((1, H))],
    out_specs=[pl.BlockSpec((NCHUNK, BM, CHUNK), lambda i: (0, i, 0)),
               _row_block(H)],
    out_shape=[jax.ShapeDtypeStruct((NCHUNK, NP, CHUNK), jnp.float32),
               jax.ShapeDtypeStruct((NP, H), jnp.float32)],
)

_comb_call = pl.pallas_call(
    _comb_body,
    grid=(_GRID,),
    in_specs=[_row_block(H), _row_block(H),
              pl.BlockSpec((NCHUNK, BM, CHUNK), lambda i: (0, i, 0)),
              _row_block(EW), _full((1, H)), _full((1, H))],
    out_specs=_row_block(H),
    out_shape=jax.ShapeDtypeStruct((NP, H), jnp.float32),
)

_read_call = pl.pallas_call(
    _read_body,
    grid=(_GRID,),
    in_specs=[_row_block(H), _full((H, H // 2)), _full((1, H // 2)),
              _full((H // 2, 128)), _full((1, 128))],
    out_specs=_row_block(128),
    out_shape=jax.ShapeDtypeStruct((NP, 128), jnp.float32),
)


def kernel(x, edge_index, edge_attr, enc_W, enc_b, enc_g, enc_beta,
           Wl, bl, Wr, ln_g, ln_b, rW1, rb1, rW2, rb2):
    f32 = jnp.float32
    pad = EPAD - E
    spread = N + jnp.arange(pad, dtype=jnp.int32) % (NP - N)
    row = jnp.concatenate([edge_index[0], jnp.zeros((pad,), jnp.int32)])
    col = jnp.concatenate([edge_index[1], spread])
    ridx = row.reshape(NSUB, NWIN, WIN)
    cidx = col.reshape(NSUB, NWIN, WIN)
    ea = jnp.concatenate(
        [edge_attr, jnp.ones((E, 1), f32), jnp.zeros((E, 111), f32)], axis=1)
    ea = jnp.concatenate([ea, jnp.zeros((pad, 128), f32)], axis=0)
    xp = jnp.pad(x, ((0, NP - N), (0, 0)))
    z128 = jnp.zeros((NP, CHUNK), f32)

    acc32 = _sc_edge_stats(ea, cidx, z128)[:, :EW]
    h = _enc_call(xp, enc_W, enc_b.reshape(1, H), enc_g.reshape(1, H),
                  enc_beta.reshape(1, H))
    for i in range(L):
        m4, r = _fused_call(h, acc32, Wl[i, :H], Wl[i, H:], Wr[i, :H],
                            Wr[i, H:], bl[i].reshape(1, H))
        s4 = _sc_segsum(m4, ridx, cidx, z128)
        h = _comb_call(h, r, s4, acc32, ln_g[i].reshape(1, H),
                       ln_b[i].reshape(1, H))
    w2p = jnp.pad(rW2, ((0, 0), (0, 126)))
    b2p = jnp.pad(rb2, (0, 126)).reshape(1, 128)
    yp = _read_call(h, rW1, rb1.reshape(1, H // 2), w2p, b2p)
    return yp[:N, :2]
